# baseline (speedup 1.0000x reference)
import jax
import jax.numpy as jnp
import numpy as np
from jax import lax
from jax.experimental import pallas as pl
from jax.experimental.pallas import tpu as pltpu
from jax.experimental.pallas import tpu_sc as plsc

N = 10000
E = 320000
IN_C = 128
OUT_C = 32
HEADS = 4
R = 8
NEG_SLOPE = 0.2
HO = HEADS * OUT_C
W = HO + 16

NC = 2
NS = 16
NW = NC * NS
CH = 64
NCHUNK = 158
EWP = NCHUNK * CH
EP = NW * EWP
NPAD = 10240
NPS = NPAD // NS


def _tc_transform_kernel(x_ref, w_ref, pt_ref, qr_ref, kr_ref,
                         xt_ref, qt_ref):
    xb = x_ref[...]
    xts, qts = [], []
    for r in range(R):
        wr = w_ref[r]
        wint = jnp.dot(wr, pt_ref[...], preferred_element_type=jnp.float32)
        wk = jnp.dot(wr, kr_ref[...], preferred_element_type=jnp.float32)
        wq = jnp.dot(wr, qr_ref[...], preferred_element_type=jnp.float32)
        xt_int = jnp.dot(xb, wint, preferred_element_type=jnp.float32)
        krow = jnp.dot(xb, wk, preferred_element_type=jnp.float32)
        xts.append(jnp.concatenate([xt_int, krow], axis=1))
        qts.append(jnp.dot(xb, wq, preferred_element_type=jnp.float32))
    xt_ref[...] = jnp.stack(xts, axis=1)
    qt_ref[...] = jnp.stack(qts, axis=1)


def _tc_index_kernel(src_ref, dst_ref, et_ref, iq_ref, ik_ref):
    t = et_ref[...]
    iq_ref[...] = dst_ref[...] * R + t
    ik_ref[...] = src_ref[...] * R + t


def _sc_edge_kernel(meta_hbm, qtab_ref, xtab_ref,
                    acc_out,
                    acc_s,
                    mv0, mv1, mv2, mv3, mv4, mv5,
                    qb0, xb0, qb1, xb1, qb2, xb2,
                    semg0, semg1, semg2, sems0, sems1, sems2,
                    semm0, semm1, semm2, semm3, semm4, semm5):
    c = lax.axis_index("c")
    s = lax.axis_index("s")
    wid = s * NC + c
    n0 = s * NPS

    zv = jnp.zeros((16,), jnp.float32)
    slots = ((qb0, xb0, semg0, sems0),
             (qb1, xb1, semg1, sems1),
             (qb2, xb2, semg2, sems2))
    metas = ((mv0, semm0), (mv1, semm1), (mv2, semm2),
             (mv3, semm3), (mv4, semm4), (mv5, semm5))

    def _zrow(e, _):
        for j in range(W // 16):
            xb0[e, pl.ds(j * 16, 16)] = zv
        return _

    lax.fori_loop(0, CH, _zrow, 0)
    for kk in range(NPS // CH):
        pltpu.sync_copy(xb0, acc_s.at[pl.ds(n0 + kk * CH, CH)])
    plsc.subcore_barrier()

    def _meta_fetch(ci, m):
        mv, semm = metas[m]
        pltpu.async_copy(meta_hbm.at[wid, ci], mv, semm)

    def _meta_wait(m):
        mv, semm = metas[m]
        pltpu.make_async_copy(meta_hbm.at[wid, 0], mv, semm).wait()

    def _fire(ci, slot, m, drain):
        qb, xb, semg, sems = slots[slot]
        mv, semm = metas[m]
        _meta_wait(m)

        def _dr():
            pltpu.make_async_copy(xb, acc_s.at[mv.at[2]], sems).wait()

        if drain is None:
            pass
        elif drain is True:
            _dr()
        else:
            pl.when(drain)(_dr)
        pltpu.async_copy(qtab_ref.at[mv.at[0]], qb, semg)
        pltpu.async_copy(xtab_ref.at[mv.at[1]], xb, semg)

    def _consume(ci, slot, m):
        qb, xb, semg, sems = slots[slot]
        mv, semm = metas[m]
        pltpu.make_async_copy(qtab_ref.at[mv.at[0]], qb, semg).wait()
        pltpu.make_async_copy(xtab_ref.at[mv.at[1]], xb, semg).wait()
        nv = E - (wid * EWP + ci * CH)
        tl = pl.ds(HO, 16)

        def _edge_fast(e, _):
            srow = qb[e] + xb[e, tl]
            a = jnp.exp(jnp.maximum(srow, srow * NEG_SLOPE))
            xb[e, tl] = a
            for j in range(HO // 16):
                sl = pl.ds(j * 16, 16)
                xb[e, sl] = xb[e, sl] * a
            return _

        def _edge_masked(e, _):
            srow = qb[e] + xb[e, tl]
            live = (e < nv).astype(jnp.float32)
            a = jnp.exp(jnp.maximum(srow, srow * NEG_SLOPE)) * live
            xb[e, tl] = a
            for j in range(HO // 16):
                sl = pl.ds(j * 16, 16)
                xb[e, sl] = xb[e, sl] * a
            return _

        @pl.when(nv >= CH)
        def _():
            lax.fori_loop(0, CH, _edge_fast, 0)

        @pl.when(nv < CH)
        def _():
            lax.fori_loop(0, CH, _edge_masked, 0)

        pltpu.async_copy(xb, acc_s.at[mv.at[2]], sems, add=True)

    for j in range(5):
        _meta_fetch(j, j)
    _fire(0, 0, 0, None)
    _fire(1, 1, 1, None)

    def _six(cb, carry):
        c0 = cb * 6
        for j in range(6):
            cc = c0 + j

            @pl.when(cc < NCHUNK)
            def _step(cc=cc, j=j):
                _consume(cc, j % 3, j % 6)

                @pl.when(cc + 2 < NCHUNK)
                def _s1():
                    _fire(cc + 2, (j + 2) % 3, (j + 2) % 6, cc >= 1)

                @pl.when(cc + 5 < NCHUNK)
                def _s2():
                    _meta_fetch(cc + 5, (j + 5) % 6)

        return carry

    lax.fori_loop(0, (NCHUNK + 5) // 6, _six, 0)
    for cc in (NCHUNK - 3, NCHUNK - 2, NCHUNK - 1):
        qb, xb, semg, sems = slots[cc % 3]
        mv, semm = metas[cc % 6]
        pltpu.make_async_copy(xb, acc_s.at[mv.at[2]], sems).wait()
    plsc.subcore_barrier()

    pltpu.sync_copy(acc_s.at[pl.ds(n0, NPS)], acc_out.at[c, pl.ds(n0, NPS)])


def _tc_finalize_kernel(acc_ref, p_ref, qm_ref, b_ref, out_ref):
    asum = acc_ref[0] + acc_ref[1]
    std = jnp.dot(asum[:, :HO], p_ref[...], preferred_element_type=jnp.float32)
    dstd = jnp.dot(asum[:, HO:], qm_ref[...], preferred_element_type=jnp.float32)
    out_ref[...] = std / (dstd + 1e-16) + b_ref[...]


def _build_perms():
    pt = np.zeros((HO, HO), np.float32)
    for h in range(HEADS):
        for cc in range(OUT_C):
            pt[h * OUT_C + cc, cc * HEADS + h] = 1.0
    p = pt.T
    qm = np.zeros((16, HO), np.float32)
    for h in range(HEADS):
        for cc in range(OUT_C):
            qm[h, h * OUT_C + cc] = 1.0
    return pt, p, qm


_PT_NP, _P_NP, _QM_NP = _build_perms()


@jax.jit
def kernel(x, edge_index, edge_type, weight, q, k, bias):
    src = jnp.pad(edge_index[0], (0, EP - E)).reshape(EP // 128, 128)
    dst = jnp.pad(edge_index[1], (0, EP - E)).reshape(EP // 128, 128)
    etp = jnp.pad(edge_type, (0, EP - E)).reshape(EP // 128, 128)
    qrep = jnp.tile(q, (1, 16 // HEADS))
    krep = jnp.tile(k, (1, 16 // HEADS))
    pt = jnp.asarray(_PT_NP)
    p = jnp.asarray(_P_NP)
    qm = jnp.asarray(_QM_NP)
    bias2d = bias.reshape(1, HO)

    nb = 1000
    xtab, qtab = pl.pallas_call(
        _tc_transform_kernel,
        grid=(N // nb,),
        in_specs=[
            pl.BlockSpec((nb, IN_C), lambda i: (i, 0)),
            pl.BlockSpec((R, IN_C, HO), lambda i: (0, 0, 0)),
            pl.BlockSpec((HO, HO), lambda i: (0, 0)),
            pl.BlockSpec((IN_C, 16), lambda i: (0, 0)),
            pl.BlockSpec((IN_C, 16), lambda i: (0, 0)),
        ],
        out_specs=[
            pl.BlockSpec((nb, R, W), lambda i: (i, 0, 0)),
            pl.BlockSpec((nb, R, 16), lambda i: (i, 0, 0)),
        ],
        out_shape=[
            jax.ShapeDtypeStruct((N, R, W), jnp.float32),
            jax.ShapeDtypeStruct((N, R, 16), jnp.float32),
        ],
    )(x, weight, pt, qrep, krep)
    xtab = xtab.reshape(N * R, W)
    qtab = qtab.reshape(N * R, 16)

    ib = 632
    idxq, idxk = pl.pallas_call(
        _tc_index_kernel,
        grid=(EP // 128 // ib,),
        in_specs=[pl.BlockSpec((ib, 128), lambda i: (i, 0))] * 3,
        out_specs=[pl.BlockSpec((ib, 128), lambda i: (i, 0))] * 2,
        out_shape=[jax.ShapeDtypeStruct((EP // 128, 128), jnp.int32)] * 2,
    )(src, dst, etp)
    meta3 = jnp.stack([idxq.reshape(NW, NCHUNK, CH),
                       idxk.reshape(NW, NCHUNK, CH),
                       dst.reshape(NW, NCHUNK, CH)], axis=2)

    mesh = plsc.VectorSubcoreMesh(core_axis_name="c", subcore_axis_name="s",
                                  num_cores=NC, num_subcores=NS)
    sc_edge = pl.kernel(
        _sc_edge_kernel,
        out_type=jax.ShapeDtypeStruct((NC, NPAD, W), jnp.float32),
        mesh=mesh,
        scratch_types=(
            pltpu.VMEM_SHARED((NPAD, W), jnp.float32),
        ) + tuple(pltpu.VMEM((3, CH), jnp.int32) for _ in range(6))
        + (
            pltpu.VMEM((CH, 16), jnp.float32),
            pltpu.VMEM((CH, W), jnp.float32),
            pltpu.VMEM((CH, 16), jnp.float32),
            pltpu.VMEM((CH, W), jnp.float32),
            pltpu.VMEM((CH, 16), jnp.float32),
            pltpu.VMEM((CH, W), jnp.float32),
        ) + tuple(pltpu.SemaphoreType.DMA for _ in range(12)),
        compiler_params=pltpu.CompilerParams(use_tc_tiling_on_sc=False),
    )
    acc_parts = sc_edge(meta3, qtab, xtab)

    grid_c = (N // nb,)
    out = pl.pallas_call(
        _tc_finalize_kernel,
        grid=grid_c,
        in_specs=[
            pl.BlockSpec((NC, nb, W), lambda i: (0, i, 0)),
            pl.BlockSpec((HO, HO), lambda i: (0, 0)),
            pl.BlockSpec((16, HO), lambda i: (0, 0)),
            pl.BlockSpec((1, HO), lambda i: (0, 0)),
        ],
        out_specs=pl.BlockSpec((nb, HO), lambda i: (i, 0)),
        out_shape=jax.ShapeDtypeStruct((N, HO), jnp.float32),
    )(acc_parts, p, qm, bias2d)

    return out


# speedup vs baseline: 1.0393x; 1.0393x over previous
import jax
import jax.numpy as jnp
import numpy as np
from jax import lax
from jax.experimental import pallas as pl
from jax.experimental.pallas import tpu as pltpu
from jax.experimental.pallas import tpu_sc as plsc

N = 10000
E = 320000
IN_C = 128
OUT_C = 32
HEADS = 4
R = 8
NEG_SLOPE = 0.2
HO = HEADS * OUT_C
W = HO + 16

NC = 2
NS = 16
NW = NC * NS
CH = 64
NCHUNK = 158
EWP = NCHUNK * CH
EP = NW * EWP
NPAD = 10240
NPS = NPAD // NS


def _tc_transform_kernel(x_ref, w_ref, pt_ref, qr_ref, kr_ref,
                         xt_ref, qt_ref):
    xt_std = jnp.dot(x_ref[...], w_ref[0], preferred_element_type=jnp.float32)
    xt_int = jnp.dot(xt_std, pt_ref[...], preferred_element_type=jnp.float32)
    krow = jnp.dot(xt_std, kr_ref[...], preferred_element_type=jnp.float32)
    xt_ref[...] = jnp.concatenate([xt_int, krow], axis=1)
    qt_ref[...] = jnp.dot(xt_std, qr_ref[...], preferred_element_type=jnp.float32)


def _tc_index_kernel(src_ref, dst_ref, et_ref, iq_ref, ik_ref):
    t = et_ref[...] * N
    iq_ref[...] = t + dst_ref[...]
    ik_ref[...] = t + src_ref[...]


def _sc_edge_kernel(meta_hbm, qtab_ref, xtab_ref,
                    acc_out,
                    acc_s,
                    mv0, mv1, mv2, mv3, mv4, mv5,
                    qb0, xb0, qb1, xb1, qb2, xb2,
                    semg0, semg1, semg2, sems0, sems1, sems2,
                    semm0, semm1, semm2, semm3, semm4, semm5):
    c = lax.axis_index("c")
    s = lax.axis_index("s")
    wid = s * NC + c
    n0 = s * NPS

    zv = jnp.zeros((16,), jnp.float32)
    slots = ((qb0, xb0, semg0, sems0),
             (qb1, xb1, semg1, sems1),
             (qb2, xb2, semg2, sems2))
    metas = ((mv0, semm0), (mv1, semm1), (mv2, semm2),
             (mv3, semm3), (mv4, semm4), (mv5, semm5))

    def _zrow(e, _):
        for j in range(W // 16):
            xb0[e, pl.ds(j * 16, 16)] = zv
        return _

    lax.fori_loop(0, CH, _zrow, 0)
    for kk in range(NPS // CH):
        pltpu.sync_copy(xb0, acc_s.at[pl.ds(n0 + kk * CH, CH)])
    plsc.subcore_barrier()

    def _meta_fetch(ci, m):
        mv, semm = metas[m]
        pltpu.async_copy(meta_hbm.at[wid, ci], mv, semm)

    def _meta_wait(m):
        mv, semm = metas[m]
        pltpu.make_async_copy(meta_hbm.at[wid, 0], mv, semm).wait()

    def _fire(ci, slot, m, drain):
        qb, xb, semg, sems = slots[slot]
        mv, semm = metas[m]
        _meta_wait(m)

        def _dr():
            pltpu.make_async_copy(xb, acc_s.at[mv.at[2]], sems).wait()

        if drain is None:
            pass
        elif drain is True:
            _dr()
        else:
            pl.when(drain)(_dr)
        pltpu.async_copy(qtab_ref.at[mv.at[0]], qb, semg)
        pltpu.async_copy(xtab_ref.at[mv.at[1]], xb, semg)

    def _consume(ci, slot, m):
        qb, xb, semg, sems = slots[slot]
        mv, semm = metas[m]
        pltpu.make_async_copy(qtab_ref.at[mv.at[0]], qb, semg).wait()
        pltpu.make_async_copy(xtab_ref.at[mv.at[1]], xb, semg).wait()
        nv = E - (wid * EWP + ci * CH)
        tl = pl.ds(HO, 16)

        def _edge_fast(e, _):
            srow = qb[e] + xb[e, tl]
            a = jnp.exp(jnp.maximum(srow, srow * NEG_SLOPE))
            xb[e, tl] = a
            for j in range(HO // 16):
                sl = pl.ds(j * 16, 16)
                xb[e, sl] = xb[e, sl] * a
            return _

        def _edge_masked(e, _):
            srow = qb[e] + xb[e, tl]
            live = (e < nv).astype(jnp.float32)
            a = jnp.exp(jnp.maximum(srow, srow * NEG_SLOPE)) * live
            xb[e, tl] = a
            for j in range(HO // 16):
                sl = pl.ds(j * 16, 16)
                xb[e, sl] = xb[e, sl] * a
            return _

        @pl.when(nv >= CH)
        def _():
            lax.fori_loop(0, CH, _edge_fast, 0)

        @pl.when(nv < CH)
        def _():
            lax.fori_loop(0, CH, _edge_masked, 0)

        pltpu.async_copy(xb, acc_s.at[mv.at[2]], sems, add=True)

    for j in range(5):
        _meta_fetch(j, j)
    _fire(0, 0, 0, None)
    _fire(1, 1, 1, None)

    def _six(cb, carry):
        c0 = cb * 6
        for j in range(6):
            cc = c0 + j

            @pl.when(cc < NCHUNK)
            def _step(cc=cc, j=j):
                _consume(cc, j % 3, j % 6)

                @pl.when(cc + 2 < NCHUNK)
                def _s1():
                    _fire(cc + 2, (j + 2) % 3, (j + 2) % 6, cc >= 1)

                @pl.when(cc + 5 < NCHUNK)
                def _s2():
                    _meta_fetch(cc + 5, (j + 5) % 6)

        return carry

    lax.fori_loop(0, (NCHUNK + 5) // 6, _six, 0)
    for cc in (NCHUNK - 3, NCHUNK - 2, NCHUNK - 1):
        qb, xb, semg, sems = slots[cc % 3]
        mv, semm = metas[cc % 6]
        pltpu.make_async_copy(xb, acc_s.at[mv.at[2]], sems).wait()
    plsc.subcore_barrier()

    pltpu.sync_copy(acc_s.at[pl.ds(n0, NPS)], acc_out.at[c, pl.ds(n0, NPS)])


def _tc_finalize_kernel(acc_ref, p_ref, qm_ref, b_ref, out_ref):
    asum = acc_ref[0] + acc_ref[1]
    std = jnp.dot(asum[:, :HO], p_ref[...], preferred_element_type=jnp.float32)
    dstd = jnp.dot(asum[:, HO:], qm_ref[...], preferred_element_type=jnp.float32)
    out_ref[...] = std / (dstd + 1e-16) + b_ref[...]


def _build_perms():
    pt = np.zeros((HO, HO), np.float32)
    for h in range(HEADS):
        for cc in range(OUT_C):
            pt[h * OUT_C + cc, cc * HEADS + h] = 1.0
    p = pt.T
    qm = np.zeros((16, HO), np.float32)
    for h in range(HEADS):
        for cc in range(OUT_C):
            qm[h, h * OUT_C + cc] = 1.0
    return pt, p, qm


_PT_NP, _P_NP, _QM_NP = _build_perms()


@jax.jit
def kernel(x, edge_index, edge_type, weight, q, k, bias):
    src = jnp.pad(edge_index[0], (0, EP - E)).reshape(EP // 128, 128)
    dst = jnp.pad(edge_index[1], (0, EP - E)).reshape(EP // 128, 128)
    etp = jnp.pad(edge_type, (0, EP - E)).reshape(EP // 128, 128)
    qrep = jnp.tile(q, (1, 16 // HEADS))
    krep = jnp.tile(k, (1, 16 // HEADS))
    pt = jnp.asarray(_PT_NP)
    p = jnp.asarray(_P_NP)
    qm = jnp.asarray(_QM_NP)
    bias2d = bias.reshape(1, HO)

    nb = 1000
    grid_a = (R, N // nb)
    xtab, qtab = pl.pallas_call(
        _tc_transform_kernel,
        grid=grid_a,
        in_specs=[
            pl.BlockSpec((nb, IN_C), lambda r, i: (i, 0)),
            pl.BlockSpec((1, IN_C, HO), lambda r, i: (r, 0, 0)),
            pl.BlockSpec((HO, HO), lambda r, i: (0, 0)),
            pl.BlockSpec((IN_C, 16), lambda r, i: (0, 0)),
            pl.BlockSpec((IN_C, 16), lambda r, i: (0, 0)),
        ],
        out_specs=[
            pl.BlockSpec((nb, W), lambda r, i: (r * (N // nb) + i, 0)),
            pl.BlockSpec((nb, 16), lambda r, i: (r * (N // nb) + i, 0)),
        ],
        out_shape=[
            jax.ShapeDtypeStruct((R * N, W), jnp.float32),
            jax.ShapeDtypeStruct((R * N, 16), jnp.float32),
        ],
    )(x, weight, pt, qrep, krep)

    ib = 632
    idxq, idxk = pl.pallas_call(
        _tc_index_kernel,
        grid=(EP // 128 // ib,),
        in_specs=[pl.BlockSpec((ib, 128), lambda i: (i, 0))] * 3,
        out_specs=[pl.BlockSpec((ib, 128), lambda i: (i, 0))] * 2,
        out_shape=[jax.ShapeDtypeStruct((EP // 128, 128), jnp.int32)] * 2,
    )(src, dst, etp)
    meta3 = jnp.stack([idxq.reshape(NW, NCHUNK, CH),
                       idxk.reshape(NW, NCHUNK, CH),
                       dst.reshape(NW, NCHUNK, CH)], axis=2)

    mesh = plsc.VectorSubcoreMesh(core_axis_name="c", subcore_axis_name="s",
                                  num_cores=NC, num_subcores=NS)
    sc_edge = pl.kernel(
        _sc_edge_kernel,
        out_type=jax.ShapeDtypeStruct((NC, NPAD, W), jnp.float32),
        mesh=mesh,
        scratch_types=(
            pltpu.VMEM_SHARED((NPAD, W), jnp.float32),
        ) + tuple(pltpu.VMEM((3, CH), jnp.int32) for _ in range(6))
        + (
            pltpu.VMEM((CH, 16), jnp.float32),
            pltpu.VMEM((CH, W), jnp.float32),
            pltpu.VMEM((CH, 16), jnp.float32),
            pltpu.VMEM((CH, W), jnp.float32),
            pltpu.VMEM((CH, 16), jnp.float32),
            pltpu.VMEM((CH, W), jnp.float32),
        ) + tuple(pltpu.SemaphoreType.DMA for _ in range(12)),
        compiler_params=pltpu.CompilerParams(use_tc_tiling_on_sc=False),
    )
    acc_parts = sc_edge(meta3, qtab, xtab)

    grid_c = (N // nb,)
    out = pl.pallas_call(
        _tc_finalize_kernel,
        grid=grid_c,
        in_specs=[
            pl.BlockSpec((NC, nb, W), lambda i: (0, i, 0)),
            pl.BlockSpec((HO, HO), lambda i: (0, 0)),
            pl.BlockSpec((16, HO), lambda i: (0, 0)),
            pl.BlockSpec((1, HO), lambda i: (0, 0)),
        ],
        out_specs=pl.BlockSpec((nb, HO), lambda i: (i, 0)),
        out_shape=jax.ShapeDtypeStruct((N, HO), jnp.float32),
    )(acc_parts, p, qm, bias2d)

    return out
